# baseline (speedup 1.0000x reference)
import functools

import numpy as np
import jax
import jax.numpy as jnp
from jax import lax
from jax.experimental import pallas as pl
from jax.experimental.pallas import tpu as pltpu
from jax.experimental.pallas import tpu_sc as plsc

N = 20000
N_CLASS = 80
PROPOSAL_COUNT = 100
IOU_THRESHOLD = 0.5
SCORE_THRESHOLD = 0.05
PERF = 5000
CLIP_RATIO = 16.0 / 1000.0
NEG = -1e9
MAX_RATIO = float(np.abs(np.log(CLIP_RATIO)))

KPAD = 5120
ROWS = 80
OUTP = 128
TW = 128

_INTERPRET = False


def _pre_kernel(t_ref, out_ref):
    T = t_ref[...]
    sc = T[:, 88:89]
    prod = sc * T
    c = jax.lax.broadcasted_iota(jnp.int32, prod.shape, 1)
    p0 = prod[:, 0:1]
    mx = jnp.max(jnp.where((c >= 1) & (c < N_CLASS), prod, NEG), axis=1,
                 keepdims=True)
    maxs = jnp.maximum(p0, mx)
    av = ((T[:, 80:81] >= 0.0) & (T[:, 81:82] >= 0.0)
          & (T[:, 82:83] <= 1.0) & (T[:, 83:84] <= 1.0))
    valid = av & (maxs >= SCORE_THRESHOLD) & (mx > p0)
    out_ref[...] = jnp.where(valid, maxs, NEG)


def _decode(x1, y1, x2, y2, dx, dy, dw, dh):
    dw = jnp.clip(dw, -MAX_RATIO, MAX_RATIO)
    dh = jnp.clip(dh, -MAX_RATIO, MAX_RATIO)
    w = x2 - x1
    h = y2 - y1
    cx = x1 + 0.5 * w
    cy = y1 + 0.5 * h
    ncx = cx + dx * w
    ncy = cy + dy * h
    nw = w * jnp.exp(dw)
    nh = h * jnp.exp(dh)
    bx1 = jnp.clip(ncx - 0.5 * nw, 0.0, 1.0)
    by1 = jnp.clip(ncy - 0.5 * nh, 0.0, 1.0)
    bx2 = jnp.clip(ncx + 0.5 * nw, 0.0, 1.0)
    by2 = jnp.clip(ncy + 0.5 * nh, 0.0, 1.0)
    areas = jnp.maximum(bx2 - bx1, 0.0) * jnp.maximum(by2 - by1, 0.0)
    return bx1, by1, bx2, by2, areas


def _main_kernel(gt_ref, g_ref, tv_ref, outl_ref, outb_ref):
    GT = gt_ref[...]
    tv = tv_ref[...]

    srow = GT[88:89, :]
    rowi = jax.lax.broadcasted_iota(jnp.int32, (ROWS, KPAD), 0)
    coli = jax.lax.broadcasted_iota(jnp.int32, (ROWS, KPAD), 1)
    S0 = jnp.where((rowi < ROWS - 1) & (tv > NEG / 2),
                   srow * GT[1:1 + ROWS, :], NEG)

    bx1, by1, bx2, by2, areas = _decode(
        GT[80:81, :], GT[81:82, :], GT[82:83, :], GT[83:84, :],
        GT[84:85, :], GT[85:86, :], GT[86:87, :], GT[87:88, :])

    G = g_ref[...]
    cx1, cy1, cx2, cy2, careas = _decode(
        G[:, 80:81], G[:, 81:82], G[:, 82:83], G[:, 83:84],
        G[:, 84:85], G[:, 85:86], G[:, 86:87], G[:, 87:88])
    C5 = jnp.concatenate([cx1, cy1, cx2, cy2, careas], axis=1)

    colsel = jax.lax.broadcasted_iota(jnp.int32, (ROWS, OUTP), 1)

    def nms_step(t, carry):
        S, selv, seli = carry
        maxv = jnp.max(S, axis=1, keepdims=True)
        m = S == maxv
        idx = jnp.min(jnp.where(m, coli, 1 << 30), axis=1, keepdims=True)
        oh = coli == idx

        R = jnp.dot(jnp.where(oh, 1.0, 0.0), C5,
                    preferred_element_type=jnp.float32)
        bx1b = R[:, 0:1]
        by1b = R[:, 1:2]
        bx2b = R[:, 2:3]
        by2b = R[:, 3:4]
        ab = R[:, 4:5]
        ix = jnp.maximum(jnp.minimum(bx2, bx2b) - jnp.maximum(bx1, bx1b),
                         0.0)
        iy = jnp.maximum(jnp.minimum(by2, by2b) - jnp.maximum(by1, by1b),
                         0.0)
        inter = ix * iy
        union = areas + ab - inter
        supp = (union > 0) & (inter / jnp.maximum(union, 1e-12)
                              > IOU_THRESHOLD)
        S = jnp.where(supp | oh, NEG, S)
        tm = colsel == t
        selv = jnp.where(tm, maxv, selv)
        seli = jnp.where(tm, idx.astype(jnp.float32), seli)
        return S, selv, seli

    init = (S0,
            jnp.full((ROWS, OUTP), NEG, dtype=jnp.float32),
            jnp.zeros((ROWS, OUTP), dtype=jnp.float32))
    _, selv, seli = jax.lax.fori_loop(0, PROPOSAL_COUNT, nms_step, init)

    fk = (jax.lax.broadcasted_iota(jnp.int32, (ROWS, OUTP), 0) * OUTP
          + colsel)
    r128 = jax.lax.broadcasted_iota(jnp.int32, (OUTP, 1), 0)

    def fin_step(t, carry):
        FL, CIDX, FV = carry
        g = jnp.max(FL, axis=(0, 1), keepdims=True)
        m = FL == g
        k = jnp.min(jnp.where(m, fk, 1 << 30), axis=(0, 1), keepdims=True)
        oh2 = fk == k
        ci = jnp.sum(jnp.where(oh2, seli, 0.0), axis=(0, 1), keepdims=True)
        fvalid = (g > NEG / 2).astype(jnp.float32)
        rowm = r128 == t
        CIDX = jnp.where(rowm, ci, CIDX)
        FV = jnp.where(rowm, fvalid, FV)
        FL = jnp.where(oh2, NEG, FL)
        return FL, CIDX, FV

    _, CIDX, FV = jax.lax.fori_loop(
        0, PROPOSAL_COUNT, fin_step,
        (selv,
         jnp.zeros((OUTP, 1), dtype=jnp.float32),
         jnp.zeros((OUTP, 1), dtype=jnp.float32)))

    coli128 = jax.lax.broadcasted_iota(jnp.int32, (OUTP, KPAD), 1)
    OHB = jnp.where((coli128 == CIDX.astype(jnp.int32)) & (FV > 0.0),
                    1.0, 0.0)

    OHL = OHB * srow
    outl_ref[...] = jnp.dot(OHL, G, preferred_element_type=jnp.float32)

    def pickb(v):
        return jnp.sum(OHB * v, axis=1, keepdims=True)

    outb_ref[...] = jnp.concatenate(
        [pickb(bx1), pickb(by1), pickb(bx2), pickb(by2)], axis=1)


def _prepass(table):
    blk = 1000
    grid = (N // blk,)
    return pl.pallas_call(
        _pre_kernel,
        grid=grid,
        in_specs=[pl.BlockSpec((blk, TW), lambda i: (i, 0))],
        out_specs=pl.BlockSpec((blk, 1), lambda i: (i, 0)),
        out_shape=jax.ShapeDtypeStruct((N, 1), jnp.float32),
        interpret=_INTERPRET,
    )(table)


def _sc_gather(table, idx):
    info = plsc.get_sparse_core_info()
    nw = info.num_cores * info.num_subcores
    bpw = KPAD // nw
    mesh = plsc.VectorSubcoreMesh(core_axis_name="c", subcore_axis_name="s")

    @functools.partial(
        pl.kernel, mesh=mesh,
        out_type=jax.ShapeDtypeStruct((KPAD, TW), jnp.float32),
        scratch_types=[
            pltpu.VMEM((bpw,), jnp.int32),
            pltpu.VMEM((bpw, TW), jnp.float32),
            pltpu.SemaphoreType.DMA,
        ],
    )
    def k(table_hbm, idx_hbm, out_hbm, idx_v, rows_v, sem):
        wid = lax.axis_index("s") * info.num_cores + lax.axis_index("c")
        base = wid * bpw
        pltpu.sync_copy(idx_hbm.at[pl.ds(base, bpw)], idx_v)
        pltpu.async_copy(table_hbm.at[idx_v], rows_v, sem).wait()
        pltpu.sync_copy(rows_v, out_hbm.at[pl.ds(base, bpw)])

    return k(table, idx)


def _main(GT, G, tv):
    return pl.pallas_call(
        _main_kernel,
        out_shape=(
            jax.ShapeDtypeStruct((OUTP, TW), jnp.float32),
            jax.ShapeDtypeStruct((OUTP, 4), jnp.float32),
        ),
        interpret=_INTERPRET,
    )(GT, G, tv)


@jax.jit
def kernel(score, logits, regress, anchors):
    s = score[0]
    lg = logits[0]
    rg = regress[0]

    table = jnp.concatenate(
        [lg, anchors, rg, s, jnp.zeros((N, TW - 89), jnp.float32)], axis=1)
    masked = _prepass(table)[:, 0]
    top_vals, top_idx = jax.lax.top_k(masked, PERF)

    idx_p = jnp.pad(top_idx, (0, KPAD - PERF))
    G = _sc_gather(table, idx_p)
    GT = G.T
    tv = jnp.pad(top_vals, (0, KPAD - PERF), constant_values=NEG)[None, :]

    outl, outb = _main(GT, G, tv)
    return (outl[None, :PROPOSAL_COUNT, :N_CLASS],
            outb[None, :PROPOSAL_COUNT, :])


# speedup vs baseline: 1.0023x; 1.0023x over previous
import functools

import numpy as np
import jax
import jax.numpy as jnp
from jax import lax
from jax.experimental import pallas as pl
from jax.experimental.pallas import tpu as pltpu
from jax.experimental.pallas import tpu_sc as plsc

N = 20000
N_CLASS = 80
PROPOSAL_COUNT = 100
IOU_THRESHOLD = 0.5
SCORE_THRESHOLD = 0.05
PERF = 5000
CLIP_RATIO = 16.0 / 1000.0
NEG = -1e9
MAX_RATIO = float(np.abs(np.log(CLIP_RATIO)))

KPAD = 5120
ROWS = 80
OUTP = 128
TW = 128

_INTERPRET = False


def _pre_kernel(t_ref, out_ref):
    T = t_ref[...]
    sc = T[:, 88:89]
    prod = sc * T
    c = jax.lax.broadcasted_iota(jnp.int32, prod.shape, 1)
    p0 = prod[:, 0:1]
    mx = jnp.max(jnp.where((c >= 1) & (c < N_CLASS), prod, NEG), axis=1,
                 keepdims=True)
    maxs = jnp.maximum(p0, mx)
    av = ((T[:, 80:81] >= 0.0) & (T[:, 81:82] >= 0.0)
          & (T[:, 82:83] <= 1.0) & (T[:, 83:84] <= 1.0))
    valid = av & (maxs >= SCORE_THRESHOLD) & (mx > p0)
    out_ref[...] = jnp.where(valid, maxs, NEG)


def _decode(x1, y1, x2, y2, dx, dy, dw, dh):
    dw = jnp.clip(dw, -MAX_RATIO, MAX_RATIO)
    dh = jnp.clip(dh, -MAX_RATIO, MAX_RATIO)
    w = x2 - x1
    h = y2 - y1
    cx = x1 + 0.5 * w
    cy = y1 + 0.5 * h
    ncx = cx + dx * w
    ncy = cy + dy * h
    nw = w * jnp.exp(dw)
    nh = h * jnp.exp(dh)
    bx1 = jnp.clip(ncx - 0.5 * nw, 0.0, 1.0)
    by1 = jnp.clip(ncy - 0.5 * nh, 0.0, 1.0)
    bx2 = jnp.clip(ncx + 0.5 * nw, 0.0, 1.0)
    by2 = jnp.clip(ncy + 0.5 * nh, 0.0, 1.0)
    areas = jnp.maximum(bx2 - bx1, 0.0) * jnp.maximum(by2 - by1, 0.0)
    return bx1, by1, bx2, by2, areas


def _main_kernel(gt_ref, g_ref, tv_ref, outl_ref, outb_ref):
    GT = gt_ref[...]
    tv = tv_ref[...]

    srow = GT[88:89, :]
    rowi = jax.lax.broadcasted_iota(jnp.int32, (ROWS, KPAD), 0)
    coli = jax.lax.broadcasted_iota(jnp.int32, (ROWS, KPAD), 1)
    S0 = jnp.where((rowi < ROWS - 1) & (tv > NEG / 2),
                   srow * GT[1:1 + ROWS, :], NEG)

    bx1, by1, bx2, by2, areas = _decode(
        GT[80:81, :], GT[81:82, :], GT[82:83, :], GT[83:84, :],
        GT[84:85, :], GT[85:86, :], GT[86:87, :], GT[87:88, :])

    G = g_ref[...]
    cx1, cy1, cx2, cy2, careas = _decode(
        G[:, 80:81], G[:, 81:82], G[:, 82:83], G[:, 83:84],
        G[:, 84:85], G[:, 85:86], G[:, 86:87], G[:, 87:88])
    C5 = jnp.concatenate([cx1, cy1, cx2, cy2, careas], axis=1)

    colsel = jax.lax.broadcasted_iota(jnp.int32, (ROWS, OUTP), 1)

    def nms_step(t, carry):
        S, selv, seli = carry
        maxv = jnp.max(S, axis=1, keepdims=True)
        m = S == maxv
        idx = jnp.min(jnp.where(m, coli, 1 << 30), axis=1, keepdims=True)
        oh = coli == idx

        R = jnp.dot(jnp.where(oh, 1.0, 0.0), C5,
                    preferred_element_type=jnp.float32)
        bx1b = R[:, 0:1]
        by1b = R[:, 1:2]
        bx2b = R[:, 2:3]
        by2b = R[:, 3:4]
        ab = R[:, 4:5]
        ix = jnp.maximum(jnp.minimum(bx2, bx2b) - jnp.maximum(bx1, bx1b),
                         0.0)
        iy = jnp.maximum(jnp.minimum(by2, by2b) - jnp.maximum(by1, by1b),
                         0.0)
        inter = ix * iy
        union = areas + ab - inter
        supp = (union > 0) & (inter / jnp.maximum(union, 1e-12)
                              > IOU_THRESHOLD)
        S = jnp.where(supp | oh, NEG, S)
        tm = colsel == t
        selv = jnp.where(tm, maxv, selv)
        seli = jnp.where(tm, idx.astype(jnp.float32), seli)
        return S, selv, seli

    init = (S0,
            jnp.full((ROWS, OUTP), NEG, dtype=jnp.float32),
            jnp.zeros((ROWS, OUTP), dtype=jnp.float32))
    _, selv, seli = jax.lax.fori_loop(0, PROPOSAL_COUNT, nms_step, init)

    fk = (jax.lax.broadcasted_iota(jnp.int32, (ROWS, OUTP), 0) * OUTP
          + colsel)
    r128 = jax.lax.broadcasted_iota(jnp.int32, (OUTP, 1), 0)

    def fin_step(t, carry):
        FL, CIDX, FV = carry
        g = jnp.max(FL, axis=(0, 1), keepdims=True)
        m = FL == g
        k = jnp.min(jnp.where(m, fk, 1 << 30), axis=(0, 1), keepdims=True)
        oh2 = fk == k
        ci = jnp.sum(jnp.where(oh2, seli, 0.0), axis=(0, 1), keepdims=True)
        fvalid = (g > NEG / 2).astype(jnp.float32)
        rowm = r128 == t
        CIDX = jnp.where(rowm, ci, CIDX)
        FV = jnp.where(rowm, fvalid, FV)
        FL = jnp.where(oh2, NEG, FL)
        return FL, CIDX, FV

    _, CIDX, FV = jax.lax.fori_loop(
        0, PROPOSAL_COUNT, fin_step,
        (selv,
         jnp.zeros((OUTP, 1), dtype=jnp.float32),
         jnp.zeros((OUTP, 1), dtype=jnp.float32)))

    coli128 = jax.lax.broadcasted_iota(jnp.int32, (OUTP, KPAD), 1)
    OHB = jnp.where((coli128 == CIDX.astype(jnp.int32)) & (FV > 0.0),
                    1.0, 0.0)

    OHL = OHB * srow
    outl_ref[...] = jnp.dot(OHL, G, preferred_element_type=jnp.float32)

    def pickb(v):
        return jnp.sum(OHB * v, axis=1, keepdims=True)

    outb_ref[...] = jnp.concatenate(
        [pickb(bx1), pickb(by1), pickb(bx2), pickb(by2)], axis=1)


def _prepass(table):
    blk = 2000
    grid = (N // blk,)
    return pl.pallas_call(
        _pre_kernel,
        grid=grid,
        in_specs=[pl.BlockSpec((blk, TW), lambda i: (i, 0))],
        out_specs=pl.BlockSpec((blk, 1), lambda i: (i, 0)),
        out_shape=jax.ShapeDtypeStruct((N, 1), jnp.float32),
        interpret=_INTERPRET,
    )(table)


def _sc_gather(table, idx):
    info = plsc.get_sparse_core_info()
    nw = info.num_cores * info.num_subcores
    bpw = KPAD // nw
    mesh = plsc.VectorSubcoreMesh(core_axis_name="c", subcore_axis_name="s")

    @functools.partial(
        pl.kernel, mesh=mesh,
        out_type=jax.ShapeDtypeStruct((KPAD, TW), jnp.float32),
        scratch_types=[
            pltpu.VMEM((bpw,), jnp.int32),
            pltpu.VMEM((bpw, TW), jnp.float32),
            pltpu.SemaphoreType.DMA,
        ],
    )
    def k(table_hbm, idx_hbm, out_hbm, idx_v, rows_v, sem):
        wid = lax.axis_index("s") * info.num_cores + lax.axis_index("c")
        base = wid * bpw
        pltpu.sync_copy(idx_hbm.at[pl.ds(base, bpw)], idx_v)
        pltpu.async_copy(table_hbm.at[idx_v], rows_v, sem).wait()
        pltpu.sync_copy(rows_v, out_hbm.at[pl.ds(base, bpw)])

    return k(table, idx)


def _main(GT, G, tv):
    return pl.pallas_call(
        _main_kernel,
        out_shape=(
            jax.ShapeDtypeStruct((OUTP, TW), jnp.float32),
            jax.ShapeDtypeStruct((OUTP, 4), jnp.float32),
        ),
        interpret=_INTERPRET,
    )(GT, G, tv)


@jax.jit
def kernel(score, logits, regress, anchors):
    s = score[0]
    lg = logits[0]
    rg = regress[0]

    table = jnp.concatenate(
        [lg, anchors, rg, s, jnp.zeros((N, TW - 89), jnp.float32)], axis=1)
    masked = _prepass(table)[:, 0]
    top_vals, top_idx = jax.lax.top_k(masked, PERF)

    idx_p = jnp.pad(top_idx, (0, KPAD - PERF))
    G = _sc_gather(table, idx_p)
    GT = G.T
    tv = jnp.pad(top_vals, (0, KPAD - PERF), constant_values=NEG)[None, :]

    outl, outb = _main(GT, G, tv)
    return (outl[None, :PROPOSAL_COUNT, :N_CLASS],
            outb[None, :PROPOSAL_COUNT, :])
